# baseline (speedup 1.0000x reference)
import functools

import jax
import jax.numpy as jnp
from jax import lax
from jax.experimental import pallas as pl
from jax.experimental.pallas import tpu as pltpu
from jax.experimental.pallas import tpu_sc as plsc

N = 10000
D = 128
H = 4
C = 128
CA = 144
NPAD = 10240
RPT = NPAD // 16
NW = 32
K = 64
ETOT = 320000 + N
NCH = -(-ETOT // (NW * K))
EPAD = NW * NCH * K
NCH0 = 184
NCH1 = 2 * NCH - NCH0
NCHMAX = max(NCH0, NCH1)
RB = 512
GRID = NPAD // RB
NREG = CA // 16


def _aug_write(outl_ref, outr_ref, yl, yr, valid):
    onec = jnp.where(valid, 1.0, 0.0)
    zpad = jnp.zeros((RB, CA - C - 1), jnp.float32)
    for h in range(H):
        fl = jnp.where(valid, yl[:, h * C:(h + 1) * C], 0.0)
        fr = jnp.where(valid, yr[:, h * C:(h + 1) * C], 0.0)
        outl_ref[h] = jnp.concatenate([fl, onec, zpad], axis=1)
        outr_ref[h] = fr


def _tc_in_body(x_ref, wl_ref, bl_ref, wr_ref, br_ref, outl_ref, outr_ref):
    r = pl.program_id(0)
    rows = r * RB + lax.broadcasted_iota(jnp.int32, (RB, 1), 0)
    valid = rows < N
    xb = x_ref[...]
    yl = jnp.dot(xb, wl_ref[...], preferred_element_type=jnp.float32) + bl_ref[...]
    yr = jnp.dot(xb, wr_ref[...], preferred_element_type=jnp.float32) + br_ref[...]
    _aug_write(outl_ref, outr_ref, yl, yr, valid)


def _combine_heads(np_ref, b_ref):
    nm = np_ref[0] + np_ref[1]
    acc = jnp.zeros((RB, C), jnp.float32)
    for h in range(H):
        num = nm[h, :, :C]
        den = nm[h, :, C:C + 1]
        acc = acc + num / (den + 1e-16)
    return acc * (1.0 / H) + b_ref[...]


def _tc_mid_body(np_ref, b1_ref, wl_ref, bl_ref, wr_ref, br_ref, outl_ref, outr_ref):
    r = pl.program_id(0)
    rows = r * RB + lax.broadcasted_iota(jnp.int32, (RB, 1), 0)
    valid = rows < N
    hb = jnp.maximum(_combine_heads(np_ref, b1_ref), 0.0)
    hb = jnp.where(valid, hb, 0.0)
    yl = jnp.dot(hb, wl_ref[...], preferred_element_type=jnp.float32) + bl_ref[...]
    yr = jnp.dot(hb, wr_ref[...], preferred_element_type=jnp.float32) + br_ref[...]
    _aug_write(outl_ref, outr_ref, yl, yr, valid)


def _tc_out_body(np_ref, b2_ref, out_ref):
    out_ref[...] = _combine_heads(np_ref, b2_ref)


def _tc_in():
    return pl.pallas_call(
        _tc_in_body,
        grid=(GRID,),
        in_specs=[
            pl.BlockSpec((RB, D), lambda r: (r, 0)),
            pl.BlockSpec((D, H * C), lambda r: (0, 0)),
            pl.BlockSpec((1, H * C), lambda r: (0, 0)),
            pl.BlockSpec((D, H * C), lambda r: (0, 0)),
            pl.BlockSpec((1, H * C), lambda r: (0, 0)),
        ],
        out_specs=[
            pl.BlockSpec((H, RB, CA), lambda r: (0, r, 0)),
            pl.BlockSpec((H, RB, C), lambda r: (0, r, 0)),
        ],
        out_shape=[jax.ShapeDtypeStruct((H, NPAD, CA), jnp.float32),
                   jax.ShapeDtypeStruct((H, NPAD, C), jnp.float32)],
    )


def _tc_mid():
    return pl.pallas_call(
        _tc_mid_body,
        grid=(GRID,),
        in_specs=[
            pl.BlockSpec((2, H, RB, CA), lambda r: (0, 0, r, 0)),
            pl.BlockSpec((1, C), lambda r: (0, 0)),
            pl.BlockSpec((C, H * C), lambda r: (0, 0)),
            pl.BlockSpec((1, H * C), lambda r: (0, 0)),
            pl.BlockSpec((C, H * C), lambda r: (0, 0)),
            pl.BlockSpec((1, H * C), lambda r: (0, 0)),
        ],
        out_specs=[
            pl.BlockSpec((H, RB, CA), lambda r: (0, r, 0)),
            pl.BlockSpec((H, RB, C), lambda r: (0, r, 0)),
        ],
        out_shape=[jax.ShapeDtypeStruct((H, NPAD, CA), jnp.float32),
                   jax.ShapeDtypeStruct((H, NPAD, C), jnp.float32)],
    )


def _tc_out():
    return pl.pallas_call(
        _tc_out_body,
        grid=(GRID,),
        in_specs=[
            pl.BlockSpec((2, H, RB, CA), lambda r: (0, 0, r, 0)),
            pl.BlockSpec((1, C), lambda r: (0, 0)),
        ],
        out_specs=pl.BlockSpec((RB, C), lambda r: (r, 0)),
        out_shape=jax.ShapeDtypeStruct((NPAD, C), jnp.float32),
    )


def _sc_body(xl_ref, xr_ref, src_ref, dst_ref, att_ref, z_ref, out_ref,
             idx_s0, idx_d0, idx_s1, idx_d1, a0, a1, b0, b1, attv, num_sh,
             sem_i0, sem_i1, sem_a0, sem_a1, sem_b0, sem_b1):
    cid = lax.axis_index("c")
    sid = lax.axis_index("s")
    wid = sid * 2 + cid
    pltpu.sync_copy(att_ref, attv)
    srcw = src_ref.at[wid]
    dstw = dst_ref.at[wid]

    nch_half = jnp.where(cid == 0, NCH0 // 2, NCH1 // 2)

    for h in range(H):
        pltpu.sync_copy(z_ref, num_sh.at[pl.ds(sid * RPT, RPT)])
        plsc.subcore_barrier()
        att_regs = [attv[h, pl.ds(r * 16, 16)] for r in range(C // 16)]
        xlh = xl_ref.at[h]
        xrh = xr_ref.at[h]

        def compute_scatter(abuf, bbuf, idxd, att_regs=att_regs):
            @plsc.parallel_loop(0, K, unroll=2)
            def edge(i):
                acc = jnp.zeros((16,), jnp.float32)
                for r in range(C // 16):
                    m = abuf[i, pl.ds(r * 16, 16)] + bbuf[i, pl.ds(r * 16, 16)]
                    m = jnp.maximum(m, m * 0.2)
                    acc = acc + m * att_regs[r]
                ev = jnp.exp(jnp.broadcast_to(plsc.cumsum(acc)[15], (16,)))
                for r in range(NREG):
                    abuf[i, pl.ds(r * 16, 16)] = abuf[i, pl.ds(r * 16, 16)] * ev

            pltpu.sync_copy(abuf, num_sh.at[idxd], add=True)

        pltpu.sync_copy(srcw.at[0], idx_s0)
        pltpu.sync_copy(dstw.at[0], idx_d0)
        pltpu.async_copy(xlh.at[idx_s0], a0, sem_a0)
        pltpu.async_copy(xrh.at[idx_d0], b0, sem_b0)
        pltpu.async_copy(srcw.at[1], idx_s1, sem_i1)
        pltpu.async_copy(dstw.at[1], idx_d1, sem_i1)

        def pair(t, carry, xlh=xlh, xrh=xrh, compute_scatter=compute_scatter):
            je = 2 * t
            more = t + 1 < nch_half
            pltpu.make_async_copy(srcw.at[je + 1], idx_s1, sem_i1).wait()
            pltpu.make_async_copy(dstw.at[je + 1], idx_d1, sem_i1).wait()
            pltpu.async_copy(xlh.at[idx_s1], a1, sem_a1)
            pltpu.async_copy(xrh.at[idx_d1], b1, sem_b1)
            pltpu.make_async_copy(xlh.at[idx_s0], a0, sem_a0).wait()
            pltpu.make_async_copy(xrh.at[idx_d0], b0, sem_b0).wait()
            compute_scatter(a0, b0, idx_d0)

            @pl.when(more)
            def _():
                pltpu.async_copy(srcw.at[je + 2], idx_s0, sem_i0)
                pltpu.async_copy(dstw.at[je + 2], idx_d0, sem_i0)

            @pl.when(more)
            def _():
                pltpu.make_async_copy(srcw.at[je + 2], idx_s0, sem_i0).wait()
                pltpu.make_async_copy(dstw.at[je + 2], idx_d0, sem_i0).wait()
                pltpu.async_copy(xlh.at[idx_s0], a0, sem_a0)
                pltpu.async_copy(xrh.at[idx_d0], b0, sem_b0)

            pltpu.make_async_copy(xlh.at[idx_s1], a1, sem_a1).wait()
            pltpu.make_async_copy(xrh.at[idx_d1], b1, sem_b1).wait()
            compute_scatter(a1, b1, idx_d1)

            @pl.when(more)
            def _():
                pltpu.async_copy(srcw.at[je + 3], idx_s1, sem_i1)
                pltpu.async_copy(dstw.at[je + 3], idx_d1, sem_i1)

            return carry

        lax.fori_loop(0, nch_half, pair, 0)
        plsc.subcore_barrier()
        pltpu.sync_copy(num_sh.at[pl.ds(sid * RPT, RPT)],
                        out_ref.at[cid].at[h].at[pl.ds(sid * RPT, RPT)])
        plsc.subcore_barrier()


@functools.lru_cache(maxsize=None)
def _sc_edge():
    mesh = plsc.VectorSubcoreMesh(core_axis_name="c", subcore_axis_name="s")
    return pl.kernel(
        _sc_body,
        out_type=jax.ShapeDtypeStruct((2, H, NPAD, CA), jnp.float32),
        mesh=mesh,
        compiler_params=pltpu.CompilerParams(needs_layout_passes=False,
                                             use_tc_tiling_on_sc=False),
        scratch_types=[
            pltpu.VMEM((K,), jnp.int32),
            pltpu.VMEM((K,), jnp.int32),
            pltpu.VMEM((K,), jnp.int32),
            pltpu.VMEM((K,), jnp.int32),
            pltpu.VMEM((K, CA), jnp.float32),
            pltpu.VMEM((K, CA), jnp.float32),
            pltpu.VMEM((K, C), jnp.float32),
            pltpu.VMEM((K, C), jnp.float32),
            pltpu.VMEM((H, C), jnp.float32),
            pltpu.VMEM_SHARED((NPAD, CA), jnp.float32),
            pltpu.SemaphoreType.DMA,
            pltpu.SemaphoreType.DMA,
            pltpu.SemaphoreType.DMA,
            pltpu.SemaphoreType.DMA,
            pltpu.SemaphoreType.DMA,
            pltpu.SemaphoreType.DMA,
        ],
    )


def kernel(x, edge_index, Wl1, bl1, Wr1, br1, att1, bias1,
           Wl2, bl2, Wr2, br2, att2, bias2):
    loop = jnp.arange(N, dtype=edge_index.dtype)
    src = jnp.concatenate([edge_index[0], loop])
    dst = jnp.concatenate([edge_index[1], loop])
    pad = EPAD - ETOT
    fill = jnp.full((pad,), N, dtype=src.dtype)
    src = jnp.concatenate([src, fill])
    dst = jnp.concatenate([dst, fill])

    def worker_layout(flat):
        rows, off = [], 0
        for w in range(NW):
            s = NCH0 if w % 2 == 0 else NCH1
            blk = flat[off:off + s * K].reshape(s, K)
            if s < NCHMAX:
                blk = jnp.concatenate(
                    [blk, jnp.full((NCHMAX - s, K), N, flat.dtype)], axis=0)
            rows.append(blk)
            off += s * K
        return jnp.stack(rows)

    srcw = worker_layout(src)
    dstw = worker_layout(dst)

    x_pad = jnp.pad(x, ((0, NPAD - N), (0, 0)))
    zblk = jnp.zeros((RPT, CA), jnp.float32)

    xl1, xr1 = _tc_in()(x_pad, Wl1, bl1.reshape(1, -1), Wr1, br1.reshape(1, -1))
    np1 = _sc_edge()(xl1, xr1, srcw, dstw, att1, zblk)
    xl2, xr2 = _tc_mid()(np1, bias1.reshape(1, -1), Wl2, bl2.reshape(1, -1),
                         Wr2, br2.reshape(1, -1))
    np2 = _sc_edge()(xl2, xr2, srcw, dstw, att2, zblk)
    outp = _tc_out()(np2, bias2.reshape(1, -1))
    return outp[:N]


# speedup vs baseline: 1.0182x; 1.0182x over previous
import functools

import jax
import jax.numpy as jnp
from jax import lax
from jax.experimental import pallas as pl
from jax.experimental.pallas import tpu as pltpu
from jax.experimental.pallas import tpu_sc as plsc

N = 10000
D = 128
H = 4
C = 128
CA = 144
NPAD = 10240
RPT = NPAD // 16
NW = 32
K = 64
ETOT = 320000 + N
NCH = -(-ETOT // (NW * K))
EPAD = NW * NCH * K
NCH0 = 180
NCH1 = 2 * NCH - NCH0
NCHMAX = max(NCH0, NCH1)
RB = 512
GRID = NPAD // RB
NREG = CA // 16


def _aug_write(outl_ref, outr_ref, yl, yr, valid):
    onec = jnp.where(valid, 1.0, 0.0)
    zpad = jnp.zeros((RB, CA - C - 1), jnp.float32)
    for h in range(H):
        fl = jnp.where(valid, yl[:, h * C:(h + 1) * C], 0.0)
        fr = jnp.where(valid, yr[:, h * C:(h + 1) * C], 0.0)
        outl_ref[h] = jnp.concatenate([fl, onec, zpad], axis=1)
        outr_ref[h] = fr


def _tc_in_body(x_ref, wl_ref, bl_ref, wr_ref, br_ref, outl_ref, outr_ref):
    r = pl.program_id(0)
    rows = r * RB + lax.broadcasted_iota(jnp.int32, (RB, 1), 0)
    valid = rows < N
    xb = x_ref[...]
    yl = jnp.dot(xb, wl_ref[...], preferred_element_type=jnp.float32) + bl_ref[...]
    yr = jnp.dot(xb, wr_ref[...], preferred_element_type=jnp.float32) + br_ref[...]
    _aug_write(outl_ref, outr_ref, yl, yr, valid)


def _combine_heads(np_ref, b_ref):
    nm = np_ref[0] + np_ref[1]
    acc = jnp.zeros((RB, C), jnp.float32)
    for h in range(H):
        num = nm[h, :, :C]
        den = nm[h, :, C:C + 1]
        acc = acc + num / (den + 1e-16)
    return acc * (1.0 / H) + b_ref[...]


def _tc_mid_body(np_ref, b1_ref, wl_ref, bl_ref, wr_ref, br_ref, outl_ref, outr_ref):
    r = pl.program_id(0)
    rows = r * RB + lax.broadcasted_iota(jnp.int32, (RB, 1), 0)
    valid = rows < N
    hb = jnp.maximum(_combine_heads(np_ref, b1_ref), 0.0)
    hb = jnp.where(valid, hb, 0.0)
    yl = jnp.dot(hb, wl_ref[...], preferred_element_type=jnp.float32) + bl_ref[...]
    yr = jnp.dot(hb, wr_ref[...], preferred_element_type=jnp.float32) + br_ref[...]
    _aug_write(outl_ref, outr_ref, yl, yr, valid)


def _tc_out_body(np_ref, b2_ref, out_ref):
    out_ref[...] = _combine_heads(np_ref, b2_ref)


def _tc_in():
    return pl.pallas_call(
        _tc_in_body,
        grid=(GRID,),
        in_specs=[
            pl.BlockSpec((RB, D), lambda r: (r, 0)),
            pl.BlockSpec((D, H * C), lambda r: (0, 0)),
            pl.BlockSpec((1, H * C), lambda r: (0, 0)),
            pl.BlockSpec((D, H * C), lambda r: (0, 0)),
            pl.BlockSpec((1, H * C), lambda r: (0, 0)),
        ],
        out_specs=[
            pl.BlockSpec((H, RB, CA), lambda r: (0, r, 0)),
            pl.BlockSpec((H, RB, C), lambda r: (0, r, 0)),
        ],
        out_shape=[jax.ShapeDtypeStruct((H, NPAD, CA), jnp.float32),
                   jax.ShapeDtypeStruct((H, NPAD, C), jnp.float32)],
    )


def _tc_mid():
    return pl.pallas_call(
        _tc_mid_body,
        grid=(GRID,),
        in_specs=[
            pl.BlockSpec((2, H, RB, CA), lambda r: (0, 0, r, 0)),
            pl.BlockSpec((1, C), lambda r: (0, 0)),
            pl.BlockSpec((C, H * C), lambda r: (0, 0)),
            pl.BlockSpec((1, H * C), lambda r: (0, 0)),
            pl.BlockSpec((C, H * C), lambda r: (0, 0)),
            pl.BlockSpec((1, H * C), lambda r: (0, 0)),
        ],
        out_specs=[
            pl.BlockSpec((H, RB, CA), lambda r: (0, r, 0)),
            pl.BlockSpec((H, RB, C), lambda r: (0, r, 0)),
        ],
        out_shape=[jax.ShapeDtypeStruct((H, NPAD, CA), jnp.float32),
                   jax.ShapeDtypeStruct((H, NPAD, C), jnp.float32)],
    )


def _tc_out():
    return pl.pallas_call(
        _tc_out_body,
        grid=(GRID,),
        in_specs=[
            pl.BlockSpec((2, H, RB, CA), lambda r: (0, 0, r, 0)),
            pl.BlockSpec((1, C), lambda r: (0, 0)),
        ],
        out_specs=pl.BlockSpec((RB, C), lambda r: (r, 0)),
        out_shape=jax.ShapeDtypeStruct((NPAD, C), jnp.float32),
    )


def _sc_body(xl_ref, xr_ref, src_ref, dst_ref, att_ref, z_ref, out_ref,
             idx_s0, idx_d0, idx_s1, idx_d1, a0, a1, b0, b1, attv, num_sh,
             sem_i0, sem_i1, sem_a0, sem_a1, sem_b0, sem_b1):
    cid = lax.axis_index("c")
    sid = lax.axis_index("s")
    wid = sid * 2 + cid
    pltpu.sync_copy(att_ref, attv)
    srcw = src_ref.at[wid]
    dstw = dst_ref.at[wid]

    nch_half = jnp.where(cid == 0, NCH0 // 2, NCH1 // 2)

    for h in range(H):
        pltpu.sync_copy(z_ref, num_sh.at[pl.ds(sid * RPT, RPT)])
        plsc.subcore_barrier()
        att_regs = [attv[h, pl.ds(r * 16, 16)] for r in range(C // 16)]
        xlh = xl_ref.at[h]
        xrh = xr_ref.at[h]

        def compute_scatter(abuf, bbuf, idxd, att_regs=att_regs):
            @plsc.parallel_loop(0, K, unroll=2)
            def edge(i):
                acc = jnp.zeros((16,), jnp.float32)
                for r in range(C // 16):
                    m = abuf[i, pl.ds(r * 16, 16)] + bbuf[i, pl.ds(r * 16, 16)]
                    m = jnp.maximum(m, m * 0.2)
                    acc = acc + m * att_regs[r]
                ev = jnp.exp(jnp.broadcast_to(plsc.cumsum(acc)[15], (16,)))
                for r in range(NREG):
                    abuf[i, pl.ds(r * 16, 16)] = abuf[i, pl.ds(r * 16, 16)] * ev

            pltpu.sync_copy(abuf, num_sh.at[idxd], add=True)

        pltpu.sync_copy(srcw.at[0], idx_s0)
        pltpu.sync_copy(dstw.at[0], idx_d0)
        pltpu.async_copy(xlh.at[idx_s0], a0, sem_a0)
        pltpu.async_copy(xrh.at[idx_d0], b0, sem_b0)
        pltpu.async_copy(srcw.at[1], idx_s1, sem_i1)
        pltpu.async_copy(dstw.at[1], idx_d1, sem_i1)

        def pair(t, carry, xlh=xlh, xrh=xrh, compute_scatter=compute_scatter):
            je = 2 * t
            more = t + 1 < nch_half
            pltpu.make_async_copy(srcw.at[je + 1], idx_s1, sem_i1).wait()
            pltpu.make_async_copy(dstw.at[je + 1], idx_d1, sem_i1).wait()
            pltpu.async_copy(xlh.at[idx_s1], a1, sem_a1)
            pltpu.async_copy(xrh.at[idx_d1], b1, sem_b1)
            pltpu.make_async_copy(xlh.at[idx_s0], a0, sem_a0).wait()
            pltpu.make_async_copy(xrh.at[idx_d0], b0, sem_b0).wait()
            compute_scatter(a0, b0, idx_d0)

            @pl.when(more)
            def _():
                pltpu.async_copy(srcw.at[je + 2], idx_s0, sem_i0)
                pltpu.async_copy(dstw.at[je + 2], idx_d0, sem_i0)

            @pl.when(more)
            def _():
                pltpu.make_async_copy(srcw.at[je + 2], idx_s0, sem_i0).wait()
                pltpu.make_async_copy(dstw.at[je + 2], idx_d0, sem_i0).wait()
                pltpu.async_copy(xlh.at[idx_s0], a0, sem_a0)
                pltpu.async_copy(xrh.at[idx_d0], b0, sem_b0)

            pltpu.make_async_copy(xlh.at[idx_s1], a1, sem_a1).wait()
            pltpu.make_async_copy(xrh.at[idx_d1], b1, sem_b1).wait()
            compute_scatter(a1, b1, idx_d1)

            @pl.when(more)
            def _():
                pltpu.async_copy(srcw.at[je + 3], idx_s1, sem_i1)
                pltpu.async_copy(dstw.at[je + 3], idx_d1, sem_i1)

            return carry

        lax.fori_loop(0, nch_half, pair, 0)
        plsc.subcore_barrier()
        pltpu.sync_copy(num_sh.at[pl.ds(sid * RPT, RPT)],
                        out_ref.at[cid].at[h].at[pl.ds(sid * RPT, RPT)])
        plsc.subcore_barrier()


@functools.lru_cache(maxsize=None)
def _sc_edge():
    mesh = plsc.VectorSubcoreMesh(core_axis_name="c", subcore_axis_name="s")
    return pl.kernel(
        _sc_body,
        out_type=jax.ShapeDtypeStruct((2, H, NPAD, CA), jnp.float32),
        mesh=mesh,
        compiler_params=pltpu.CompilerParams(needs_layout_passes=False,
                                             use_tc_tiling_on_sc=False),
        scratch_types=[
            pltpu.VMEM((K,), jnp.int32),
            pltpu.VMEM((K,), jnp.int32),
            pltpu.VMEM((K,), jnp.int32),
            pltpu.VMEM((K,), jnp.int32),
            pltpu.VMEM((K, CA), jnp.float32),
            pltpu.VMEM((K, CA), jnp.float32),
            pltpu.VMEM((K, C), jnp.float32),
            pltpu.VMEM((K, C), jnp.float32),
            pltpu.VMEM((H, C), jnp.float32),
            pltpu.VMEM_SHARED((NPAD, CA), jnp.float32),
            pltpu.SemaphoreType.DMA,
            pltpu.SemaphoreType.DMA,
            pltpu.SemaphoreType.DMA,
            pltpu.SemaphoreType.DMA,
            pltpu.SemaphoreType.DMA,
            pltpu.SemaphoreType.DMA,
        ],
    )


def kernel(x, edge_index, Wl1, bl1, Wr1, br1, att1, bias1,
           Wl2, bl2, Wr2, br2, att2, bias2):
    loop = jnp.arange(N, dtype=edge_index.dtype)
    src = jnp.concatenate([edge_index[0], loop])
    dst = jnp.concatenate([edge_index[1], loop])
    pad = EPAD - ETOT
    fill = jnp.full((pad,), N, dtype=src.dtype)
    src = jnp.concatenate([src, fill])
    dst = jnp.concatenate([dst, fill])

    def worker_layout(flat):
        rows, off = [], 0
        for w in range(NW):
            s = NCH0 if w % 2 == 0 else NCH1
            blk = flat[off:off + s * K].reshape(s, K)
            if s < NCHMAX:
                blk = jnp.concatenate(
                    [blk, jnp.full((NCHMAX - s, K), N, flat.dtype)], axis=0)
            rows.append(blk)
            off += s * K
        return jnp.stack(rows)

    srcw = worker_layout(src)
    dstw = worker_layout(dst)

    x_pad = jnp.pad(x, ((0, NPAD - N), (0, 0)))
    zblk = jnp.zeros((RPT, CA), jnp.float32)

    xl1, xr1 = _tc_in()(x_pad, Wl1, bl1.reshape(1, -1), Wr1, br1.reshape(1, -1))
    np1 = _sc_edge()(xl1, xr1, srcw, dstw, att1, zblk)
    xl2, xr2 = _tc_mid()(np1, bias1.reshape(1, -1), Wl2, bl2.reshape(1, -1),
                         Wr2, br2.reshape(1, -1))
    np2 = _sc_edge()(xl2, xr2, srcw, dstw, att2, zblk)
    outp = _tc_out()(np2, bias2.reshape(1, -1))
    return outp[:N]
